# Initial kernel scaffold; baseline (speedup 1.0000x reference)
import functools

import jax
import jax.numpy as jnp
from jax import lax
from jax.experimental import pallas as pl
from jax.experimental.pallas import tpu as pltpu
from jax.experimental.pallas import tpu_sc as plsc

NC, NS, L = 2, 16, 16
NW = NC * NS
BATCH, SEQ = 16384, 200
N = BATCH * SEQ
NPT = N // NW
CHUNK = 4096
NCHUNK = NPT // CHUNK
VPC = 2 * CHUNK // L


def _sc_lookup_body(params_hbm, idx_hbm, out_hbm, params_v, idx_v, out_v):
    wid = lax.axis_index("s") * NC + lax.axis_index("c")
    base0 = wid * NPT

    pltpu.sync_copy(params_hbm, params_v)

    lane = lax.iota(jnp.int32, L)
    odd = (lane & 1) == 1
    half = lax.shift_right_logical(lane, 1)

    acc0 = jnp.zeros((L,), jnp.float32)
    acc1 = jnp.zeros((L,), jnp.float32)
    for k in range(4):
        e = params_v[0, pl.ds(k * L, L)]
        acc0 = acc0 + e * params_v[1, pl.ds(k * L, L)]
        acc1 = acc1 + e * params_v[2, pl.ds(k * L, L)]
    brow = params_v[3, pl.ds(0, L)]
    b0 = jnp.sum(jnp.where(lane == 0, brow, 0.0))
    b1 = jnp.sum(jnp.where(lane == 1, brow, 0.0))
    l0 = jnp.sum(acc0) + b0
    l1 = jnp.sum(acc1) + b1

    def softmax_pair(v0, v1):
        m = jnp.maximum(v0, v1)
        ev = jnp.exp(jnp.where(odd,
                               jnp.full((L,), v1 - m, jnp.float32),
                               jnp.full((L,), v0 - m, jnp.float32)))
        return ev / (jnp.sum(ev) * (2.0 / L))

    p_zero = softmax_pair(b0, b1)
    delta = softmax_pair(l0, l1) - p_zero

    def chunk_body(c, carry):
        src = base0 + c * CHUNK
        pltpu.sync_copy(idx_hbm.at[pl.ds(src, CHUNK)], idx_v)

        def vec_body(j, carry2):
            gidx = half + j * (L // 2)
            vrep = plsc.load_gather(idx_v, [gidx])
            out_v[pl.ds(j * L, L)] = p_zero + vrep.astype(jnp.float32) * delta
            return carry2

        lax.fori_loop(0, VPC, vec_body, 0, unroll=4)
        pltpu.sync_copy(out_v, out_hbm.at[pl.ds(2 * src, 2 * CHUNK)])
        return carry

    lax.fori_loop(0, NCHUNK, chunk_body, 0)


@functools.partial(
    pl.kernel,
    mesh=plsc.VectorSubcoreMesh(core_axis_name="c", subcore_axis_name="s"),
    out_type=jax.ShapeDtypeStruct((2 * N,), jnp.float32),
    scratch_types=[
        pltpu.VMEM((4, 64), jnp.float32),
        pltpu.VMEM((CHUNK,), jnp.int32),
        pltpu.VMEM((2 * CHUNK,), jnp.float32),
    ],
)
def _sc_lookup(params_hbm, idx_hbm, out_hbm, params_v, idx_v, out_v):
    _sc_lookup_body(params_hbm, idx_hbm, out_hbm, params_v, idx_v, out_v)


def kernel(x_indices, t, embedding_weight, readout_w, readout_b):
    del t
    emb1 = jnp.pad(embedding_weight[1], (0, 64 - 50))
    w0 = jnp.pad(readout_w[0], (0, 64 - 50))
    w1 = jnp.pad(readout_w[1], (0, 64 - 50))
    brow = jnp.pad(readout_b, (0, 64 - 2))
    params = jnp.stack([emb1, w0, w1, brow])
    idx_flat = x_indices.reshape(-1)
    out = _sc_lookup(params, idx_flat)
    return out.reshape(BATCH, SEQ, 2)


# speedup vs baseline: 6.2627x; 6.2627x over previous
import functools

import jax
import jax.numpy as jnp
from jax import lax
from jax.experimental import pallas as pl
from jax.experimental.pallas import tpu as pltpu
from jax.experimental.pallas import tpu_sc as plsc

NC, NS, L = 2, 16, 16
NW = NC * NS
BATCH, SEQ = 16384, 200
N = BATCH * SEQ
NPT = N // NW
CHUNK = 4096
NCHUNK = NPT // CHUNK
VPC = 2 * CHUNK // L


def _sc_lookup_body(params_hbm, idx_hbm, out_hbm, params_v, idx_v, out_v):
    wid = lax.axis_index("s") * NC + lax.axis_index("c")
    base0 = wid * NPT

    pltpu.sync_copy(params_hbm, params_v)

    lane = lax.iota(jnp.int32, L)
    odd = (lane & 1) == 1
    half = lax.shift_right_logical(lane, 1)

    prods0 = [params_v[0, pl.ds(k * L, L)] * params_v[1, pl.ds(k * L, L)]
              for k in range(4)]
    prods1 = [params_v[0, pl.ds(k * L, L)] * params_v[2, pl.ds(k * L, L)]
              for k in range(4)]
    d0 = jnp.float32(0.0)
    d1 = jnp.float32(0.0)
    for k in range(4):
        for j in range(L):
            if k * L + j < 50:
                d0 = d0 + prods0[k][j]
                d1 = d1 + prods1[k][j]
    brow = params_v[3, pl.ds(0, L)]
    b0 = brow[0]
    b1 = brow[1]
    l0 = d0 + b0
    l1 = d1 + b1

    m_z = jnp.maximum(b0, b1)
    m_o = jnp.maximum(l0, l1)
    shifted = jnp.where(lane == 0, jnp.full((L,), b0 - m_z, jnp.float32),
              jnp.where(lane == 1, jnp.full((L,), b1 - m_z, jnp.float32),
              jnp.where(lane == 2, jnp.full((L,), l0 - m_o, jnp.float32),
                        jnp.full((L,), l1 - m_o, jnp.float32))))
    evec = jnp.exp(shifted)
    ez0 = evec[0]
    ez1 = evec[1]
    eo0 = evec[2]
    eo1 = evec[3]

    ez_alt = jnp.where(odd, jnp.full((L,), ez1, jnp.float32),
                       jnp.full((L,), ez0, jnp.float32))
    eo_alt = jnp.where(odd, jnp.full((L,), eo1, jnp.float32),
                       jnp.full((L,), eo0, jnp.float32))
    p_zero = ez_alt / jnp.full((L,), ez0 + ez1, jnp.float32)
    delta = eo_alt / jnp.full((L,), eo0 + eo1, jnp.float32) - p_zero

    def chunk_body(c, carry):
        src = base0 + c * CHUNK
        pltpu.sync_copy(idx_hbm.at[pl.ds(src, CHUNK)], idx_v)

        def vec_body(j, carry2):
            gidx = half + j * (L // 2)
            vrep = plsc.load_gather(idx_v, [gidx])
            out_v[pl.ds(j * L, L)] = p_zero + vrep.astype(jnp.float32) * delta
            return carry2

        lax.fori_loop(0, VPC, vec_body, 0, unroll=4)
        pltpu.sync_copy(out_v, out_hbm.at[pl.ds(2 * src, 2 * CHUNK)])
        return carry

    lax.fori_loop(0, NCHUNK, chunk_body, 0)


@functools.partial(
    pl.kernel,
    mesh=plsc.VectorSubcoreMesh(core_axis_name="c", subcore_axis_name="s"),
    compiler_params=pltpu.CompilerParams(needs_layout_passes=False),
    out_type=jax.ShapeDtypeStruct((2 * N,), jnp.float32),
    scratch_types=[
        pltpu.VMEM((4, 64), jnp.float32),
        pltpu.VMEM((CHUNK,), jnp.int32),
        pltpu.VMEM((2 * CHUNK,), jnp.float32),
    ],
)
def _sc_lookup(params_hbm, idx_hbm, out_hbm, params_v, idx_v, out_v):
    _sc_lookup_body(params_hbm, idx_hbm, out_hbm, params_v, idx_v, out_v)


def kernel(x_indices, t, embedding_weight, readout_w, readout_b):
    del t
    emb1 = jnp.pad(embedding_weight[1], (0, 64 - 50))
    w0 = jnp.pad(readout_w[0], (0, 64 - 50))
    w1 = jnp.pad(readout_w[1], (0, 64 - 50))
    brow = jnp.pad(readout_b, (0, 64 - 2))
    params = jnp.stack([emb1, w0, w1, brow])
    idx_flat = x_indices.reshape(-1)
    out = _sc_lookup(params, idx_flat)
    return out.reshape(BATCH, SEQ, 2)
